# baseline (speedup 1.0000x reference)
import functools

import jax
import jax.numpy as jnp
from jax import lax
from jax.experimental import pallas as pl
from jax.experimental.pallas import tpu as pltpu
from jax.experimental.pallas import tpu_sc as plsc

L = 16
NC, NS = 2, 16
NW = NC * NS
BATCH = 16384
BPW = BATCH // NW
D = 32
CHUNK = 128
NCHUNK = BPW // CHUNK
GROUPS = BPW // L

_mesh = plsc.VectorSubcoreMesh(core_axis_name="c", subcore_axis_name="s")


@functools.partial(
    pl.kernel,
    out_type=jax.ShapeDtypeStruct((NW, BPW), jnp.float32),
    mesh=_mesh,
    scratch_types=[
        pltpu.VMEM((NCHUNK, CHUNK), jnp.int32),
        pltpu.VMEM((NCHUNK, CHUNK), jnp.int32),
        pltpu.VMEM((BPW, D), jnp.float32),
        pltpu.VMEM((BPW, D), jnp.float32),
        pltpu.VMEM((2 * D, L), jnp.float32),
        pltpu.VMEM((L,), jnp.float32),
        pltpu.VMEM((BPW,), jnp.float32),
        pltpu.SemaphoreType.DMA,
    ],
    compiler_params=pltpu.CompilerParams(
        needs_layout_passes=False, use_tc_tiling_on_sc=False),
)
def _sc_fwd(users_hbm, movies_hbm, ut_hbm, mt_hbm, w_hbm, b_hbm, out_hbm,
            idx_u, idx_m, u_rows, m_rows, w_v, b_v, out_v, sem):
    wid = lax.axis_index("s") * NC + lax.axis_index("c")

    pltpu.sync_copy(users_hbm.at[wid], idx_u)
    pltpu.sync_copy(movies_hbm.at[wid], idx_m)
    pltpu.sync_copy(w_hbm, w_v)
    pltpu.sync_copy(b_hbm, b_v)

    copies = []
    for j in range(NCHUNK):
        copies.append(pltpu.async_copy(
            ut_hbm.at[idx_u.at[j]], u_rows.at[pl.ds(j * CHUNK, CHUNK)], sem))
        copies.append(pltpu.async_copy(
            mt_hbm.at[idx_m.at[j]], m_rows.at[pl.ds(j * CHUNK, CHUNK)], sem))
    for c in copies:
        c.wait()

    bvec = b_v[...]
    lane = lax.iota(jnp.int32, L)
    def g_body(g, _):
        row = g * L + lane
        acc = bvec
        for d in range(D):
            cd = jnp.full((L,), d, jnp.int32)
            ucol = plsc.load_gather(u_rows, [row, cd])
            mcol = plsc.load_gather(m_rows, [row, cd])
            acc = acc + ucol * w_v[d] + mcol * w_v[D + d]
        out_v[pl.ds(g * L, L)] = acc
        return ()

    lax.fori_loop(0, GROUPS, g_body, ())
    pltpu.sync_copy(out_v, out_hbm.at[wid])


def kernel(users, movies, user_table, movie_table, W, b):
    users_r = users.astype(jnp.int32).reshape(NW, NCHUNK, CHUNK)
    movies_r = movies.astype(jnp.int32).reshape(NW, NCHUNK, CHUNK)
    w_e = jnp.broadcast_to(W.reshape(2 * D, 1), (2 * D, L))
    b16 = jnp.broadcast_to(b.reshape(1), (L,))
    out = _sc_fwd(users_r, movies_r, user_table, movie_table, w_e, b16)
    return out.reshape(BATCH, 1)
